# Initial kernel scaffold; baseline (speedup 1.0000x reference)
import jax
import jax.numpy as jnp
from jax.experimental import pallas as pl


def kernel(xyz, features, W1, b1, gamma1, beta1, W2, b2, gamma2, beta2, W3, b3, gamma3, beta3):
    raise NotImplementedError("write your pallas kernel here")


# speedup vs baseline: 1.0020x; 1.0020x over previous
import functools

import jax
import jax.numpy as jnp
import numpy as np
from jax.experimental import pallas as pl

B, N, C = 8, 8192, 128
M, K = 512, 32
RADIUS = 0.2
EPS = 1e-5
CIN0 = 256
TR = 512
R = B * M * K
GT = TR // K

_INV_SQRT2 = np.float32(1.0 / np.sqrt(2.0))


def _gelu(x):
    return 0.5 * x * (1.0 + jax.lax.erf(x * _INV_SQRT2))


def _fps_kernel(xyz_ref, nxyz_ref):
    x = xyz_ref[0]
    y = xyz_ref[1]
    z = xyz_ref[2]
    lin = jax.lax.broadcasted_iota(jnp.int32, (B, N), 1)
    lin_m = jax.lax.broadcasted_iota(jnp.int32, (B, M), 1)

    def body(i, state):
        dists, far, ax, ay, az = state
        sel = lin_m == i
        mask = lin == far
        cx = jnp.sum(jnp.where(mask, x, 0.0), axis=1, keepdims=True)
        cy = jnp.sum(jnp.where(mask, y, 0.0), axis=1, keepdims=True)
        cz = jnp.sum(jnp.where(mask, z, 0.0), axis=1, keepdims=True)
        ax = jnp.where(sel, cx, ax)
        ay = jnp.where(sel, cy, ay)
        az = jnp.where(sel, cz, az)
        dx = x - cx
        dy = y - cy
        dz = z - cz
        d = dx * dx + dy * dy + dz * dz
        dists = jnp.minimum(dists, d)
        mx = jnp.max(dists, axis=1, keepdims=True)
        far = jnp.min(jnp.where(dists == mx, lin, N), axis=1, keepdims=True)
        return dists, far, ax, ay, az

    dists0 = jnp.full((B, N), jnp.inf, dtype=jnp.float32)
    far0 = jnp.zeros((B, 1), dtype=jnp.int32)
    a0 = jnp.zeros((B, M), dtype=jnp.float32)
    _, _, ax, ay, az = jax.lax.fori_loop(0, M, body, (dists0, far0, a0, a0, a0))
    nxyz_ref[0] = ax
    nxyz_ref[1] = ay
    nxyz_ref[2] = az


def _fps(xyz_t):
    return pl.pallas_call(
        _fps_kernel,
        out_shape=jax.ShapeDtypeStruct((3, B, M), jnp.float32),
    )(xyz_t)


def _ball_query(xyz, new_xyz):
    d2 = (jnp.sum(new_xyz ** 2, -1)[:, :, None] + jnp.sum(xyz ** 2, -1)[:, None, :]
          - 2.0 * jnp.einsum('bmd,bnd->bmn', new_xyz, xyz))
    dist = jnp.sqrt(jnp.clip(d2, 0.0, None))
    dist = jnp.where(dist > RADIUS, jnp.inf, dist)
    neg_vals, idx = jax.lax.top_k(-dist, K)
    gathered = -neg_vals
    first = jnp.broadcast_to(idx[:, :, 0:1], idx.shape)
    idx = jnp.where(jnp.isinf(gathered), first, idx)
    return idx


def _layer_kernel(x_ref, w_ref, b_ref, st_ref, y_ref, ps_ref, pq_ref, *, first):
    x = x_ref[...]
    if not first:
        x = _gelu(x * st_ref[0:1, :] + st_ref[1:2, :])
    y = jnp.dot(x, w_ref[...], preferred_element_type=jnp.float32) + b_ref[0:1, :]
    y_ref[...] = y
    ps_ref[...] = jnp.sum(y, axis=0, keepdims=True)[None]
    pq_ref[...] = jnp.sum(y * y, axis=0, keepdims=True)[None]


def _layer(x, w, bias, stats, *, first):
    cin = x.shape[1]
    cout = w.shape[1]
    nt = R // TR
    y, ps, pq = pl.pallas_call(
        functools.partial(_layer_kernel, first=first),
        grid=(nt,),
        in_specs=[
            pl.BlockSpec((TR, cin), lambda i: (i, 0)),
            pl.BlockSpec((cin, cout), lambda i: (0, 0)),
            pl.BlockSpec((1, cout), lambda i: (0, 0)),
            pl.BlockSpec((2, cin), lambda i: (0, 0)),
        ],
        out_specs=[
            pl.BlockSpec((TR, cout), lambda i: (i, 0)),
            pl.BlockSpec((1, 1, cout), lambda i: (i, 0, 0)),
            pl.BlockSpec((1, 1, cout), lambda i: (i, 0, 0)),
        ],
        out_shape=[
            jax.ShapeDtypeStruct((R, cout), jnp.float32),
            jax.ShapeDtypeStruct((nt, 1, cout), jnp.float32),
            jax.ShapeDtypeStruct((nt, 1, cout), jnp.float32),
        ],
    )(x, w, bias, stats)
    return y, ps[:, 0, :], pq[:, 0, :]


def _final_kernel(x_ref, st_ref, o_ref):
    x = _gelu(x_ref[...] * st_ref[0:1, :] + st_ref[1:2, :])
    o_ref[...] = jnp.max(x.reshape(GT, K, x.shape[-1]), axis=1)


def _final(x, stats):
    cout = x.shape[1]
    ng = (B * M) // GT
    return pl.pallas_call(
        _final_kernel,
        grid=(ng,),
        in_specs=[
            pl.BlockSpec((TR, cout), lambda i: (i, 0)),
            pl.BlockSpec((2, cout), lambda i: (0, 0)),
        ],
        out_specs=pl.BlockSpec((GT, cout), lambda i: (i, 0)),
        out_shape=jax.ShapeDtypeStruct((B * M, cout), jnp.float32),
    )(x, stats)


def _make_stats(ps, pq, gamma, beta):
    mean = jnp.sum(ps, axis=0) / np.float32(R)
    var = jnp.sum(pq, axis=0) / np.float32(R) - mean * mean
    scale = gamma / jnp.sqrt(var + EPS)
    shift = beta - mean * scale
    return jnp.stack([scale, shift], axis=0)


def kernel(xyz, features, W1, b1, gamma1, beta1, W2, b2, gamma2, beta2,
           W3, b3, gamma3, beta3):
    xyz_t = jnp.transpose(xyz, (2, 0, 1))
    nxyz_t = _fps(xyz_t)
    new_xyz = jnp.transpose(nxyz_t, (1, 2, 0))

    gidx = _ball_query(xyz, new_xyz)
    flat = gidx.reshape(B, M * K)
    gxyz = jnp.take_along_axis(
        xyz, jnp.broadcast_to(flat[:, :, None], (B, M * K, 3)), axis=1
    ).reshape(B, M, K, 3)
    gxyz = gxyz - new_xyz[:, :, None, :]
    gfeat = jnp.take_along_axis(
        features, jnp.broadcast_to(flat[:, :, None], (B, M * K, C)), axis=1
    ).reshape(B, M, K, C)
    g0 = jnp.concatenate(
        [gxyz, gfeat, jnp.zeros((B, M, K, CIN0 - C - 3), jnp.float32)], axis=-1
    ).reshape(R, CIN0)

    w1 = jnp.zeros((CIN0, W1.shape[0]), jnp.float32).at[: C + 3, :].set(W1.T)
    w2 = W2.T
    w3 = W3.T

    dummy = jnp.zeros((2, CIN0), jnp.float32)
    y1, ps1, pq1 = _layer(g0, w1, b1[None, :], dummy, first=True)
    st1 = _make_stats(ps1, pq1, gamma1, beta1)
    y2, ps2, pq2 = _layer(y1, w2, b2[None, :], st1, first=False)
    st2 = _make_stats(ps2, pq2, gamma2, beta2)
    y3, ps3, pq3 = _layer(y2, w3, b3[None, :], st2, first=False)
    st3 = _make_stats(ps3, pq3, gamma3, beta3)
    new_features = _final(y3, st3).reshape(B, M, W3.shape[0])
    return new_xyz, new_features
